# baseline (speedup 1.0000x reference)
import functools

import jax
import jax.numpy as jnp
from jax import lax
from jax.experimental import pallas as pl
from jax.experimental.pallas import tpu as pltpu
from jax.experimental.pallas import tpu_sc as plsc

_N = 10000
_E = 320000
_H = 128

_NC = 2
_NS = 16
_NW = _NC * _NS
_C = 80
_CPW = _E // _C // _NW
_BLK = 64
_RPT = 640


def _segsum_body(x_hbm, src_hbm, dst_hbm, zeros_hbm, out_hbm,
                 src_v, dst_v, rows_v, acc, gsem, ssem):
    c = lax.axis_index("c")
    s = lax.axis_index("s")
    wid = s * _NC + c

    @pl.when(s < _NS - 1)
    def _():
        pltpu.sync_copy(zeros_hbm.at[pl.ds(s * _RPT, _RPT)],
                        acc.at[pl.ds(s * _RPT, _RPT)])

    @pl.when(s == _NS - 1)
    def _():
        pltpu.sync_copy(zeros_hbm.at[pl.ds(_N - 400, 400)],
                        acc.at[pl.ds(_N - 400, 400)])

    plsc.subcore_barrier()

    def issue_g(j, k):
        pltpu.async_copy(x_hbm.at[src_v.at[j]], rows_v.at[k], gsem.at[k])

    def wait_g(j, k):
        pltpu.make_async_copy(x_hbm.at[src_v.at[j]], rows_v.at[k],
                              gsem.at[k]).wait()

    def issue_s(j, k):
        pltpu.async_copy(rows_v.at[k], acc.at[dst_v.at[j]], ssem.at[k],
                         add=True)

    def wait_s(j, k):
        pltpu.make_async_copy(rows_v.at[k], acc.at[dst_v.at[j]],
                              ssem.at[k]).wait()

    def run_block(base, nch):
        pltpu.sync_copy(src_hbm.at[wid, pl.ds(base, nch)],
                        src_v.at[pl.ds(0, nch)])
        pltpu.sync_copy(dst_hbm.at[wid, pl.ds(base, nch)],
                        dst_v.at[pl.ds(0, nch)])
        issue_g(0, 0)
        wait_g(0, 0)
        issue_s(0, 0)
        issue_g(1, 1)
        wait_g(1, 1)
        issue_s(1, 1)
        issue_g(2, 2)

        def body(i, carry):
            for u in range(3):
                j = 2 + 3 * i + u
                b = (2 + u) % 3
                wait_g(j, b)
                issue_s(j, b)
                wait_s(j - 2, u % 3)
                issue_g(j + 1, u % 3)
            return carry

        t = (nch - 3) // 3
        lax.fori_loop(0, t, body, 0, unroll=False)
        for jj in range(3 * t + 2, nch):
            wait_g(jj, jj % 3)
            issue_s(jj, jj % 3)
            if jj + 1 < nch:
                wait_s(jj - 2, (jj + 1) % 3)
                issue_g(jj + 1, (jj + 1) % 3)
        for k in range(3 * t + 1, nch):
            wait_s(k, k % 3)

    run_block(0, _BLK)
    run_block(_BLK, _CPW - _BLK)

    plsc.subcore_barrier()

    @pl.when(s < _NS - 1)
    def _():
        pltpu.sync_copy(acc.at[pl.ds(s * _RPT, _RPT)],
                        out_hbm.at[c, pl.ds(s * _RPT, _RPT)])

    @pl.when(s == _NS - 1)
    def _():
        pltpu.sync_copy(acc.at[pl.ds(_N - 400, 400)],
                        out_hbm.at[c, pl.ds(_N - 400, 400)])


_segsum = functools.partial(
    pl.kernel,
    out_type=jax.ShapeDtypeStruct((_NC, _N, _H), jnp.float32),
    mesh=plsc.VectorSubcoreMesh(core_axis_name="c", subcore_axis_name="s"),
    scratch_types=[
        pltpu.VMEM((_BLK, _C), jnp.int32),
        pltpu.VMEM((_BLK, _C), jnp.int32),
        pltpu.VMEM((3, _C, _H), jnp.float32),
        pltpu.VMEM_SHARED((_N, _H), jnp.float32),
        pltpu.SemaphoreType.DMA((3,)),
        pltpu.SemaphoreType.DMA((3,)),
    ],
)(_segsum_body)


def _bn_relu(z, g, b):
    mean = jnp.mean(z, axis=0, keepdims=True)
    d = z - mean
    var = jnp.mean(d * d, axis=0, keepdims=True)
    return jnp.maximum(d * lax.rsqrt(var + 1e-5) * g + b, 0.0)


def _dense1_body(h_ref, p_ref, w_ref, b_ref, eps_ref, ga_ref, ba_ref,
                 gb_ref, bb_ref, o_ref):
    msg = p_ref[0] + p_ref[1]
    hp = (1.0 + eps_ref[0, 0]) * h_ref[...] + msg
    z = jnp.dot(hp, w_ref[...], preferred_element_type=jnp.float32) + b_ref[...]
    u = _bn_relu(z, ga_ref[...], ba_ref[...])
    o_ref[...] = _bn_relu(u, gb_ref[...], bb_ref[...])


def _dense2_body(h_ref, p_ref, w_ref, b_ref, eps_ref, ga_ref, ba_ref,
                 gb_ref, bb_ref, wu_ref, bu_ref, o_ref):
    msg = p_ref[0] + p_ref[1]
    hp = (1.0 + eps_ref[0, 0]) * h_ref[...] + msg
    z = jnp.dot(hp, w_ref[...], preferred_element_type=jnp.float32) + b_ref[...]
    u = _bn_relu(z, ga_ref[...], ba_ref[...])
    v = _bn_relu(u, gb_ref[...], bb_ref[...])
    o_ref[...] = (jnp.dot(v, wu_ref[...], preferred_element_type=jnp.float32)
                  + bu_ref[0, 0])


_dense1 = pl.pallas_call(
    _dense1_body,
    out_shape=jax.ShapeDtypeStruct((_N, _H), jnp.float32),
)

_dense2 = pl.pallas_call(
    _dense2_body,
    out_shape=jax.ShapeDtypeStruct((_N, 1), jnp.float32),
)


def kernel(x, edge_index, W1, b1, W2, b2, eps1, eps2,
           g1a, be1a, g1b, be1b, g2a, be2a, g2b, be2b, Wu, bu):
    src = edge_index[0].astype(jnp.int32).reshape(_NW, _CPW, _C)
    dst = edge_index[1].astype(jnp.int32).reshape(_NW, _CPW, _C)
    zeros = jnp.zeros((_N, _H), jnp.float32)

    row = lambda v: v.reshape(1, _H)
    sca = lambda v: v.reshape(1, 1)

    p1 = _segsum(x, src, dst, zeros)
    h1 = _dense1(x, p1, W1, row(b1), sca(eps1), row(g1a), row(be1a),
                 row(g1b), row(be1b))
    p2 = _segsum(h1, src, dst, zeros)
    scores = _dense2(h1, p2, W2, row(b2), sca(eps2), row(g2a), row(be2a),
                     row(g2b), row(be2b), Wu, sca(bu))
    return scores


# speedup vs baseline: 1.2388x; 1.2388x over previous
import functools

import jax
import jax.numpy as jnp
from jax import lax
from jax.experimental import pallas as pl
from jax.experimental.pallas import tpu as pltpu
from jax.experimental.pallas import tpu_sc as plsc

_N = 10000
_E = 320000
_H = 128

_NC = 2
_NS = 16
_NW = _NC * _NS
_C = 80
_CPW = _E // _C // _NW
_BLK = 64
_RPT = 640


def _segsum_body(x_hbm, src_hbm, dst_hbm, zeros_hbm, out_hbm,
                 src_v, dst_v, rows_v, acc, gsem):
    c = lax.axis_index("c")
    s = lax.axis_index("s")
    wid = s * _NC + c

    @pl.when(s < _NS - 1)
    def _():
        pltpu.sync_copy(zeros_hbm.at[pl.ds(s * _RPT, _RPT)],
                        acc.at[pl.ds(s * _RPT, _RPT)])

    @pl.when(s == _NS - 1)
    def _():
        pltpu.sync_copy(zeros_hbm.at[pl.ds(_N - 400, 400)],
                        acc.at[pl.ds(_N - 400, 400)])

    plsc.subcore_barrier()

    def issue_g(j, k):
        pltpu.async_copy(x_hbm.at[src_v.at[j]], rows_v.at[k], gsem.at[k])

    def wait_g(j, k):
        pltpu.make_async_copy(x_hbm.at[src_v.at[j]], rows_v.at[k],
                              gsem.at[k]).wait()

    def scatter(j, k):
        pltpu.sync_copy(rows_v.at[k], acc.at[dst_v.at[j]], add=True)

    def run_block(base, nch):
        pltpu.sync_copy(src_hbm.at[wid, pl.ds(base, nch)],
                        src_v.at[pl.ds(0, nch)])
        pltpu.sync_copy(dst_hbm.at[wid, pl.ds(base, nch)],
                        dst_v.at[pl.ds(0, nch)])
        issue_g(0, 0)

        def body(i, carry):
            j = 2 * i
            issue_g(j + 1, 1)
            wait_g(j, 0)
            scatter(j, 0)
            issue_g(j + 2, 0)
            wait_g(j + 1, 1)
            scatter(j + 1, 1)
            return carry

        pairs = (nch - 2) // 2
        lax.fori_loop(0, pairs, body, 0, unroll=False)
        j = 2 * pairs
        issue_g(j + 1, 1)
        wait_g(j, 0)
        scatter(j, 0)
        if j + 2 < nch:
            issue_g(j + 2, 0)
        wait_g(j + 1, 1)
        scatter(j + 1, 1)
        if j + 2 < nch:
            wait_g(j + 2, 0)
            scatter(j + 2, 0)

    run_block(0, _BLK)
    run_block(_BLK, _CPW - _BLK)

    plsc.subcore_barrier()

    @pl.when(s < _NS - 1)
    def _():
        pltpu.sync_copy(acc.at[pl.ds(s * _RPT, _RPT)],
                        out_hbm.at[c, pl.ds(s * _RPT, _RPT)])

    @pl.when(s == _NS - 1)
    def _():
        pltpu.sync_copy(acc.at[pl.ds(_N - 400, 400)],
                        out_hbm.at[c, pl.ds(_N - 400, 400)])


_segsum = functools.partial(
    pl.kernel,
    out_type=jax.ShapeDtypeStruct((_NC, _N, _H), jnp.float32),
    mesh=plsc.VectorSubcoreMesh(core_axis_name="c", subcore_axis_name="s"),
    scratch_types=[
        pltpu.VMEM((_BLK, _C), jnp.int32),
        pltpu.VMEM((_BLK, _C), jnp.int32),
        pltpu.VMEM((2, _C, _H), jnp.float32),
        pltpu.VMEM_SHARED((_N, _H), jnp.float32),
        pltpu.SemaphoreType.DMA((2,)),
    ],
)(_segsum_body)


def _bn_relu(z, g, b):
    mean = jnp.mean(z, axis=0, keepdims=True)
    d = z - mean
    var = jnp.mean(d * d, axis=0, keepdims=True)
    return jnp.maximum(d * lax.rsqrt(var + 1e-5) * g + b, 0.0)


def _dense1_body(h_ref, p_ref, w_ref, b_ref, eps_ref, ga_ref, ba_ref,
                 gb_ref, bb_ref, o_ref):
    msg = p_ref[0] + p_ref[1]
    hp = (1.0 + eps_ref[0, 0]) * h_ref[...] + msg
    z = jnp.dot(hp, w_ref[...], preferred_element_type=jnp.float32) + b_ref[...]
    u = _bn_relu(z, ga_ref[...], ba_ref[...])
    o_ref[...] = _bn_relu(u, gb_ref[...], bb_ref[...])


def _dense2_body(h_ref, p_ref, w_ref, b_ref, eps_ref, ga_ref, ba_ref,
                 gb_ref, bb_ref, wu_ref, bu_ref, o_ref):
    msg = p_ref[0] + p_ref[1]
    hp = (1.0 + eps_ref[0, 0]) * h_ref[...] + msg
    z = jnp.dot(hp, w_ref[...], preferred_element_type=jnp.float32) + b_ref[...]
    u = _bn_relu(z, ga_ref[...], ba_ref[...])
    v = _bn_relu(u, gb_ref[...], bb_ref[...])
    o_ref[...] = (jnp.dot(v, wu_ref[...], preferred_element_type=jnp.float32)
                  + bu_ref[0, 0])


_dense1 = pl.pallas_call(
    _dense1_body,
    out_shape=jax.ShapeDtypeStruct((_N, _H), jnp.float32),
)

_dense2 = pl.pallas_call(
    _dense2_body,
    out_shape=jax.ShapeDtypeStruct((_N, 1), jnp.float32),
)


def kernel(x, edge_index, W1, b1, W2, b2, eps1, eps2,
           g1a, be1a, g1b, be1b, g2a, be2a, g2b, be2b, Wu, bu):
    src = edge_index[0].astype(jnp.int32).reshape(_NW, _CPW, _C)
    dst = edge_index[1].astype(jnp.int32).reshape(_NW, _CPW, _C)
    zeros = jnp.zeros((_N, _H), jnp.float32)

    row = lambda v: v.reshape(1, _H)
    sca = lambda v: v.reshape(1, 1)

    p1 = _segsum(x, src, dst, zeros)
    h1 = _dense1(x, p1, W1, row(b1), sca(eps1), row(g1a), row(be1a),
                 row(g1b), row(be1b))
    p2 = _segsum(h1, src, dst, zeros)
    scores = _dense2(h1, p2, W2, row(b2), sca(eps2), row(g2a), row(be2a),
                     row(g2b), row(be2b), Wu, sca(bu))
    return scores
